# Initial kernel scaffold; baseline (speedup 1.0000x reference)
import functools

import jax
import jax.numpy as jnp
from jax import lax
from jax.experimental import pallas as pl
from jax.experimental.pallas import tpu as pltpu
from jax.experimental.pallas import tpu_sc as plsc

E = 320000
V = 10000
D = 128
L = 16
NC = 2
NS = 16
NW = NC * NS
NV_PER = 313
V_PAD = NW * NV_PER
C = 256

_NEG = jnp.float32(-jnp.inf)


def _body(x_hbm, vid_hbm, bounds_hbm, out_hbm, bounds_v, vid_v, x_v,
          out_local, sem):
    w = lax.axis_index("s") * NC + lax.axis_index("c")
    v_lo = w * NV_PER

    pltpu.sync_copy(bounds_hbm, bounds_v)
    e0 = bounds_v[w]
    e1 = bounds_v[w + 1]

    neg = jnp.full((L,), _NEG, dtype=jnp.float32)

    def init_body(t, _):
        out_local[pl.ds(t * L, L)] = neg
        return 0

    lax.fori_loop(0, NV_PER * D // L, init_body, 0)

    base0 = (e0 // 8) * 8
    nchunks = lax.select(e1 > e0, (e1 - base0 + C - 1) // C, 0)

    def chunk_body(k, carry):
        s = base0 + k * C
        b = jnp.minimum(s, E - C)
        pltpu.sync_copy(vid_hbm.at[pl.ds(b, C)], vid_v)
        pltpu.sync_copy(x_hbm.at[pl.ds(b, C)], x_v)
        lo = jnp.maximum(e0, s) - b
        hi = jnp.minimum(e1, s + C) - b

        def edge_body(i, ecarry):
            prev_vid = ecarry[0]
            run = list(ecarry[1:])
            vidi = vid_v[i]
            row = vidi - v_lo
            m = jnp.full((L,), vidi != prev_vid)
            for j in range(D // L):
                xj = x_v[i, pl.ds(j * L, L)]
                rj = jnp.maximum(jnp.where(m, neg, run[j]), xj)
                out_local[pl.ds(row * D + j * L, L)] = rj
                run[j] = rj
            return (vidi, *run)

        return lax.fori_loop(lo, hi, edge_body, carry)

    carry0 = (jnp.int32(-1),) + tuple(neg for _ in range(D // L))
    lax.fori_loop(0, nchunks, chunk_body, carry0)

    pltpu.sync_copy(out_local, out_hbm.at[pl.ds(v_lo * D, NV_PER * D)])


@jax.jit
def kernel(x_sp, vertex_id):
    vbounds = jnp.arange(NW + 1, dtype=jnp.int32) * NV_PER
    bounds = jnp.searchsorted(vertex_id, vbounds, side="left").astype(jnp.int32)
    bounds = jnp.concatenate(
        [bounds, jnp.full((64 - NW - 1,), E, dtype=jnp.int32)])

    mesh = plsc.VectorSubcoreMesh(
        core_axis_name="c", subcore_axis_name="s", num_cores=NC,
        num_subcores=NS)
    run = pl.kernel(
        _body,
        out_type=jax.ShapeDtypeStruct((V_PAD * D,), jnp.float32),
        mesh=mesh,
        scratch_types=[
            pltpu.VMEM((64,), jnp.int32),
            pltpu.VMEM((C,), jnp.int32),
            pltpu.VMEM((C, D), jnp.float32),
            pltpu.VMEM((NV_PER * D,), jnp.float32),
            pltpu.SemaphoreType.DMA,
        ],
    )
    out = run(x_sp, vertex_id, bounds)
    return out.reshape(V_PAD, D)[:V]


# speedup vs baseline: 1.2479x; 1.2479x over previous
import functools

import jax
import jax.numpy as jnp
from jax import lax
from jax.experimental import pallas as pl
from jax.experimental.pallas import tpu as pltpu
from jax.experimental.pallas import tpu_sc as plsc

E = 320000
V = 10000
D = 128
L = 16
NC = 2
NS = 16
NW = NC * NS
NV_PER = 313
V_PAD = NW * NV_PER
C = 256

_NEG = float("-inf")


def _body(x_hbm, vid_hbm, bounds_hbm, out_hbm, bounds_v, vid_v, x_v,
          out_local, sem):
    w = lax.axis_index("s") * NC + lax.axis_index("c")
    v_lo = w * NV_PER

    pltpu.sync_copy(bounds_hbm, bounds_v)
    bvec = bounds_v[pl.ds(w, L)]
    e0 = bvec[0]
    e1 = bvec[1]

    neg = jnp.full((L,), _NEG, dtype=jnp.float32)

    def init_body(t, _):
        out_local[pl.ds(t * L, L)] = neg
        return 0

    lax.fori_loop(0, NV_PER * D // L, init_body, 0)

    base0 = (e0 // 8) * 8
    nchunks = lax.select(e1 > e0, (e1 - base0 + C - 1) // C, 0)

    def chunk_body(k, carry):
        s = base0 + k * C
        b = jnp.minimum(s, E - C)
        pltpu.sync_copy(vid_hbm.at[pl.ds(b, C)], vid_v.at[pl.ds(0, C)])
        pltpu.sync_copy(x_hbm.at[pl.ds(b, C)], x_v)
        lo = jnp.maximum(e0, s) - b
        hi = jnp.minimum(e1, s + C) - b

        def edge_body(i, ecarry):
            prev_vid = ecarry[0]
            run = list(ecarry[1:])
            vidi = vid_v[pl.ds(i, L)][0]
            row = vidi - v_lo
            gate_s = jnp.where(vidi != prev_vid, _NEG, jnp.float32(jnp.inf))
            gate = jnp.full((L,), gate_s, dtype=jnp.float32)
            for j in range(D // L):
                xj = x_v[i, pl.ds(j * L, L)]
                rj = jnp.maximum(jnp.minimum(run[j], gate), xj)
                out_local[pl.ds(row * D + j * L, L)] = rj
                run[j] = rj
            return (vidi, *run)

        return lax.fori_loop(lo, hi, edge_body, carry)

    carry0 = (jnp.int32(-1),) + tuple(neg for _ in range(D // L))
    lax.fori_loop(0, nchunks, chunk_body, carry0)

    pltpu.sync_copy(out_local, out_hbm.at[pl.ds(v_lo * D, NV_PER * D)])


@jax.jit
def kernel(x_sp, vertex_id):
    vbounds = jnp.arange(NW + 1, dtype=jnp.int32) * NV_PER
    bounds = jnp.searchsorted(vertex_id, vbounds, side="left").astype(jnp.int32)
    bounds = jnp.concatenate(
        [bounds, jnp.full((64 - NW - 1,), E, dtype=jnp.int32)])

    mesh = plsc.VectorSubcoreMesh(
        core_axis_name="c", subcore_axis_name="s", num_cores=NC,
        num_subcores=NS)
    run = pl.kernel(
        _body,
        out_type=jax.ShapeDtypeStruct((V_PAD * D,), jnp.float32),
        mesh=mesh,
        scratch_types=[
            pltpu.VMEM((64,), jnp.int32),
            pltpu.VMEM((C + L,), jnp.int32),
            pltpu.VMEM((C, D), jnp.float32),
            pltpu.VMEM((NV_PER * D,), jnp.float32),
            pltpu.SemaphoreType.DMA,
        ],
    )
    out = run(x_sp, vertex_id, bounds)
    return out.reshape(V_PAD, D)[:V]


# speedup vs baseline: 2.7802x; 2.2279x over previous
import functools

import jax
import jax.numpy as jnp
from jax import lax
from jax.experimental import pallas as pl
from jax.experimental.pallas import tpu as pltpu
from jax.experimental.pallas import tpu_sc as plsc

E = 320000
V = 10000
D = 128
L = 16
NC = 2
NS = 16
NW = NC * NS
NV_PER = 313
V_PAD = NW * NV_PER
C = 256

_NEG = float("-inf")
_INF = float("inf")


def _body(x_hbm, vid_hbm, bounds_hbm, out_hbm, bounds_v, vid_v, x_v,
          out_local, sem):
    w = lax.axis_index("s") * NC + lax.axis_index("c")
    v_lo = w * NV_PER

    pltpu.sync_copy(bounds_hbm, bounds_v)
    bvec = bounds_v[pl.ds(w, L)]
    e0 = bvec[0]
    e1 = bvec[1]

    neg = jnp.full((L,), _NEG, dtype=jnp.float32)

    def init_body(t, _):
        out_local[pl.ds(t * L, L)] = neg
        return 0

    lax.fori_loop(0, NV_PER * D // L, init_body, 0)

    base0 = (e0 // 8) * 8
    nchunks = lax.select(e1 > e0, (e1 - base0 + C - 1) // C, 0)

    def chunk_body(k, carry):
        s = base0 + k * C
        b = jnp.minimum(s, E - C)
        pltpu.sync_copy(vid_hbm.at[pl.ds(b, C)], vid_v.at[pl.ds(0, C)])
        pltpu.sync_copy(x_hbm.at[pl.ds(b, C)], x_v.at[pl.ds(0, C)])
        lo = jnp.maximum(e0, s) - b
        hi = jnp.minimum(e1, s + C) - b
        gb0 = (lo // L) * L
        ng = (hi - gb0 + L - 1) // L

        def group_fn(gb, gcarry):
            prev_vid, prev_row = gcarry[0], gcarry[1]
            run = list(gcarry[2:])
            vids = vid_v[pl.ds(gb, L)]
            for t in range(L):
                vt = vids[t]
                pos = gb + t
                live = pos < hi
                valid = jnp.logical_and(pos >= lo, live)
                is_new = vt != prev_vid
                do_flush = jnp.logical_and(is_new, valid)
                frow = prev_row
                frun = list(run)

                @pl.when(do_flush)
                def _():
                    for j in range(D // L):
                        out_local[pl.ds(frow * D + j * L, L)] = frun[j]

                gate_s = jnp.where(jnp.logical_and(is_new, live), _NEG,
                                   _INF)
                vgate_s = jnp.where(live, _INF, _NEG)
                for j in range(D // L):
                    xj = x_v[pos, pl.ds(j * L, L)]
                    run[j] = jnp.maximum(jnp.minimum(run[j], gate_s),
                                         jnp.minimum(xj, vgate_s))
                prev_row = jnp.where(valid, vt - v_lo, prev_row)
                prev_vid = vt
            return (prev_vid, prev_row, *run)

        return plsc.parallel_loop(gb0, gb0 + ng * L, L,
                                  carry=carry)(group_fn)

    carry0 = (jnp.int32(-1), jnp.int32(NV_PER)) + tuple(
        neg for _ in range(D // L))
    fcarry = lax.fori_loop(0, nchunks, chunk_body, carry0)
    last_row = fcarry[1]
    for j in range(D // L):
        out_local[pl.ds(last_row * D + j * L, L)] = fcarry[2 + j]

    pltpu.sync_copy(out_local.at[pl.ds(0, NV_PER * D)],
                    out_hbm.at[pl.ds(v_lo * D, NV_PER * D)])


@jax.jit
def kernel(x_sp, vertex_id):
    vbounds = jnp.arange(NW + 1, dtype=jnp.int32) * NV_PER
    bounds = jnp.searchsorted(vertex_id, vbounds, side="left").astype(jnp.int32)
    bounds = jnp.concatenate(
        [bounds, jnp.full((64 - NW - 1,), E, dtype=jnp.int32)])

    mesh = plsc.VectorSubcoreMesh(
        core_axis_name="c", subcore_axis_name="s", num_cores=NC,
        num_subcores=NS)
    run = pl.kernel(
        _body,
        out_type=jax.ShapeDtypeStruct((V_PAD * D,), jnp.float32),
        mesh=mesh,
        scratch_types=[
            pltpu.VMEM((64,), jnp.int32),
            pltpu.VMEM((C + L,), jnp.int32),
            pltpu.VMEM((C + L, D), jnp.float32),
            pltpu.VMEM(((NV_PER + 1) * D,), jnp.float32),
            pltpu.SemaphoreType.DMA,
        ],
    )
    out = run(x_sp, vertex_id, bounds)
    return out.reshape(V_PAD, D)[:V]


# speedup vs baseline: 4.0144x; 1.4440x over previous
import functools

import jax
import jax.numpy as jnp
from jax import lax
from jax.experimental import pallas as pl
from jax.experimental.pallas import tpu as pltpu
from jax.experimental.pallas import tpu_sc as plsc

E = 320000
V = 10000
D = 128
L = 16
NC = 2
NS = 16
NW = NC * NS
NV_PER = 313
V_PAD = NW * NV_PER
C = 256

_NEG = float("-inf")
_INF = float("inf")


def _body(x_hbm, vid_hbm, bounds_hbm, out_hbm, bounds_v, vid_v0, vid_v1,
          x_v0, x_v1, out_local, sem0, sem1):
    w = lax.axis_index("s") * NC + lax.axis_index("c")
    v_lo = w * NV_PER

    pltpu.sync_copy(bounds_hbm, bounds_v)
    bvec = bounds_v[pl.ds(w, L)]
    e0 = bvec[0]
    e1 = bvec[1]

    neg = jnp.full((L,), _NEG, dtype=jnp.float32)

    def init_body(t, _):
        out_local[pl.ds(t * L, L)] = neg
        return 0

    lax.fori_loop(0, NV_PER * D // L, init_body, 0)

    base0 = (e0 // 8) * 8
    nchunks = lax.select(e1 > e0, (e1 - base0 + C - 1) // C, 0)

    def chunk_base(k):
        return jnp.minimum(base0 + k * C, E - C)

    def start_fetch(k, vid_v, x_v, sem):
        b = chunk_base(k)
        pltpu.make_async_copy(vid_hbm.at[pl.ds(b, C)],
                              vid_v.at[pl.ds(0, C)], sem).start()
        pltpu.make_async_copy(x_hbm.at[pl.ds(b, C)],
                              x_v.at[pl.ds(0, C)], sem).start()

    def wait_fetch(k, vid_v, x_v, sem):
        b = chunk_base(k)
        pltpu.make_async_copy(vid_hbm.at[pl.ds(b, C)],
                              vid_v.at[pl.ds(0, C)], sem).wait()
        pltpu.make_async_copy(x_hbm.at[pl.ds(b, C)],
                              x_v.at[pl.ds(0, C)], sem).wait()

    def process_chunk(k, carry, vid_v, x_v):
        s = base0 + k * C
        b = chunk_base(k)
        lo = jnp.maximum(e0, s) - b
        hi = jnp.minimum(e1, s + C) - b
        gb0 = (lo // L) * L
        ng = jnp.maximum((hi - gb0 + L - 1) // L, 0)

        def group_fn(gb, gcarry):
            prev_vid, prev_row = gcarry[0], gcarry[1]
            run = list(gcarry[2:])
            vids = vid_v[pl.ds(gb, L)]
            for t in range(L):
                vt = vids[t]
                pos = gb + t
                live = pos < hi
                valid = jnp.logical_and(pos >= lo, live)
                is_new = vt != prev_vid
                do_flush = jnp.logical_and(is_new, valid)
                frow = prev_row
                frun = list(run)

                @pl.when(do_flush)
                def _():
                    for j in range(D // L):
                        out_local[pl.ds(frow * D + j * L, L)] = frun[j]

                gate_s = jnp.where(jnp.logical_and(is_new, live), _NEG,
                                   _INF)
                vgate_s = jnp.where(live, _INF, _NEG)
                for j in range(D // L):
                    xj = x_v[pos, pl.ds(j * L, L)]
                    run[j] = jnp.maximum(jnp.minimum(run[j], gate_s),
                                         jnp.minimum(xj, vgate_s))
                prev_row = jnp.where(valid, vt - v_lo, prev_row)
                prev_vid = vt
            return (prev_vid, prev_row, *run)

        return plsc.parallel_loop(gb0, gb0 + ng * L, L,
                                  carry=carry)(group_fn)

    start_fetch(jnp.int32(0), vid_v0, x_v0, sem0)
    start_fetch(jnp.int32(1), vid_v1, x_v1, sem1)

    def pair_body(m, carry):
        k0 = 2 * m
        wait_fetch(k0, vid_v0, x_v0, sem0)
        carry = process_chunk(k0, carry, vid_v0, x_v0)
        start_fetch(k0 + 2, vid_v0, x_v0, sem0)
        k1 = k0 + 1
        wait_fetch(k1, vid_v1, x_v1, sem1)
        carry = process_chunk(k1, carry, vid_v1, x_v1)
        start_fetch(k1 + 2, vid_v1, x_v1, sem1)
        return carry

    npairs = (nchunks + 1) // 2
    carry0 = (jnp.int32(-1), jnp.int32(NV_PER)) + tuple(
        neg for _ in range(D // L))
    fcarry = lax.fori_loop(0, npairs, pair_body, carry0)
    wait_fetch(2 * npairs, vid_v0, x_v0, sem0)
    wait_fetch(2 * npairs + 1, vid_v1, x_v1, sem1)
    last_row = fcarry[1]
    for j in range(D // L):
        out_local[pl.ds(last_row * D + j * L, L)] = fcarry[2 + j]

    pltpu.sync_copy(out_local.at[pl.ds(0, NV_PER * D)],
                    out_hbm.at[pl.ds(v_lo * D, NV_PER * D)])


@jax.jit
def kernel(x_sp, vertex_id):
    vbounds = jnp.arange(NW + 1, dtype=jnp.int32) * NV_PER
    bounds = jnp.searchsorted(vertex_id, vbounds, side="left").astype(jnp.int32)
    bounds = jnp.concatenate(
        [bounds, jnp.full((64 - NW - 1,), E, dtype=jnp.int32)])

    mesh = plsc.VectorSubcoreMesh(
        core_axis_name="c", subcore_axis_name="s", num_cores=NC,
        num_subcores=NS)
    run = pl.kernel(
        _body,
        out_type=jax.ShapeDtypeStruct((V_PAD * D,), jnp.float32),
        mesh=mesh,
        scratch_types=[
            pltpu.VMEM((64,), jnp.int32),
            pltpu.VMEM((C + L,), jnp.int32),
            pltpu.VMEM((C + L,), jnp.int32),
            pltpu.VMEM((C + L, D), jnp.float32),
            pltpu.VMEM((C + L, D), jnp.float32),
            pltpu.VMEM(((NV_PER + 1) * D,), jnp.float32),
            pltpu.SemaphoreType.DMA,
            pltpu.SemaphoreType.DMA,
        ],
    )
    out = run(x_sp, vertex_id, bounds)
    return out.reshape(V_PAD, D)[:V]
